# baseline (speedup 1.0000x reference)
import jax
import jax.numpy as jnp
from jax.experimental import pallas as pl
from jax.experimental.pallas import tpu as pltpu

N_IN = 3072
N_HID = 2048
N_OUT = 100
BATCH = 4096
BM = 256
NB = BATCH // BM
CH = 512
_CVT_SUB = 256


def _convert_chunk(stage_ref, slot, dst_ref, dst_row):
    def body(i, _):
        sl = pl.ds(i * _CVT_SUB, _CVT_SUB)
        dst_ref[pl.ds(dst_row + i * _CVT_SUB, _CVT_SUB), :] = (
            stage_ref[slot, sl, :].astype(jnp.bfloat16)
        )
        return 0
    jax.lax.fori_loop(0, CH // _CVT_SUB, body, 0, unroll=True)


def _mlp_body(x_ref, w0_hbm, b0_ref, w1_hbm, b1_ref, w2_hbm, b2_ref,
              o_ref, w0_v, w1_v, w2_v, h0_v, stage, stage2, sems, sem2):
    i = pl.program_id(0)
    first = i == 0

    chunks = (
        [(w0_hbm, w0_v, r) for r in range(0, N_IN, CH)]
        + [(w1_hbm, w1_v, r) for r in range(0, N_HID, CH)]
    )
    n = len(chunks)

    def _dma(k):
        src, _, row = chunks[k]
        return pltpu.make_async_copy(
            src.at[pl.ds(row, CH), :], stage.at[k % 2], sems.at[k % 2]
        )

    @pl.when(first)
    def _load_weights():
        _dma(0).start()
        _dma(1).start()
        pltpu.make_async_copy(w2_hbm, stage2, sem2).start()
        for k in range(n):
            _dma(k).wait()
            _, dst, row = chunks[k]
            _convert_chunk(stage, k % 2, dst, row)
            if k + 2 < n:
                _dma(k + 2).start()
        pltpu.make_async_copy(w2_hbm, stage2, sem2).wait()
        def w2body(j, _):
            sl = pl.ds(j * _CVT_SUB, _CVT_SUB)
            w2_v[sl, :] = stage2[sl, :].astype(jnp.bfloat16)
            return 0
        jax.lax.fori_loop(0, N_HID // _CVT_SUB, w2body, 0, unroll=True)

    bf = jnp.bfloat16
    rd = i % 2
    wr = (i + 1) % 2

    hp = h0_v[rd, :, :]
    h1 = jnp.dot(hp, w1_v[...], preferred_element_type=jnp.float32)
    h1 = jnp.maximum(h1 + b1_ref[...], 0.0)
    o_ref[...] = (
        jnp.dot(h1.astype(bf), w2_v[...], preferred_element_type=jnp.float32)
        + b2_ref[...]
    )

    h0 = jnp.dot(x_ref[...].astype(bf), w0_v[...],
                 preferred_element_type=jnp.float32)
    h0_v[wr, :, :] = jnp.maximum(h0 + b0_ref[...], 0.0).astype(bf)


def kernel(x, W0, b0, W1, b1, W2, b2):
    b0r = b0.reshape(1, N_HID)
    b1r = b1.reshape(1, N_HID)
    b2r = b2.reshape(1, N_OUT)
    grid = (NB + 1,)
    return pl.pallas_call(
        _mlp_body,
        grid=grid,
        in_specs=[
            pl.BlockSpec((BM, N_IN), lambda i: (jnp.minimum(i, NB - 1), 0)),
            pl.BlockSpec(memory_space=pl.ANY),
            pl.BlockSpec((1, N_HID), lambda i: (0, 0)),
            pl.BlockSpec(memory_space=pl.ANY),
            pl.BlockSpec((1, N_HID), lambda i: (0, 0)),
            pl.BlockSpec(memory_space=pl.ANY),
            pl.BlockSpec((1, N_OUT), lambda i: (0, 0)),
        ],
        out_specs=pl.BlockSpec(
            (BM, N_OUT), lambda i: (jnp.maximum(i - 1, 0), 0)
        ),
        out_shape=jax.ShapeDtypeStruct((BATCH, N_OUT), jnp.float32),
        scratch_shapes=[
            pltpu.VMEM((N_IN, N_HID), jnp.bfloat16),
            pltpu.VMEM((N_HID, N_HID), jnp.bfloat16),
            pltpu.VMEM((N_HID, N_OUT), jnp.bfloat16),
            pltpu.VMEM((2, BM, N_HID), jnp.bfloat16),
            pltpu.VMEM((2, CH, N_HID), jnp.float32),
            pltpu.VMEM((N_HID, N_OUT), jnp.float32),
            pltpu.SemaphoreType.DMA((2,)),
            pltpu.SemaphoreType.DMA,
        ],
        compiler_params=pltpu.CompilerParams(
            dimension_semantics=("arbitrary",),
        ),
    )(x, W0, b0r, W1, b1r, W2, b2r)


# speedup vs baseline: 1.0202x; 1.0202x over previous
import jax
import jax.numpy as jnp
from jax.experimental import pallas as pl
from jax.experimental.pallas import tpu as pltpu

N_IN = 3072
N_HID = 2048
N_OUT = 100
BATCH = 4096
BM = 256
CH = 512
_CVT_SUB = 256
_NSLOT = 4

_W0_CHUNKS = N_IN // CH
_W1_CHUNKS = N_HID // CH
_N_CHUNKS = _W0_CHUNKS + _W1_CHUNKS


def _convert_chunk(stage_ref, slot, dst_ref, dst_row):
    def body(i, _):
        sl = pl.ds(i * _CVT_SUB, _CVT_SUB)
        dst_ref[pl.ds(dst_row + i * _CVT_SUB, _CVT_SUB), :] = (
            stage_ref[slot, sl, :].astype(jnp.bfloat16)
        )
        return 0
    jax.lax.fori_loop(0, CH // _CVT_SUB, body, 0, unroll=True)


def _mlp_body(x_ref, w0_hbm, b0_ref, w1_hbm, b1_ref, w2_hbm, b2_ref,
              o_ref, w0_v, w1_v, w2_v, stage, stage2, sems, sem2):
    first = pl.program_id(0) == 0

    chunks = (
        [(w0_hbm, w0_v, r) for r in range(0, N_IN, CH)]
        + [(w1_hbm, w1_v, r) for r in range(0, N_HID, CH)]
    )

    def _dma(k):
        src, _, row = chunks[k]
        return pltpu.make_async_copy(
            src.at[pl.ds(row, CH), :], stage.at[k % _NSLOT], sems.at[k % _NSLOT]
        )

    def _step(k):
        _dma(k).wait()
        _, dst, row = chunks[k]
        _convert_chunk(stage, k % _NSLOT, dst, row)
        if k + _NSLOT < _N_CHUNKS:
            _dma(k + _NSLOT).start()

    @pl.when(first)
    def _load_w0():
        for k in range(_NSLOT):
            _dma(k).start()
        pltpu.make_async_copy(w2_hbm, stage2, sem2).start()
        for k in range(_W0_CHUNKS):
            _step(k)

    bf = jnp.bfloat16
    h = jnp.dot(x_ref[...].astype(bf), w0_v[...],
                preferred_element_type=jnp.float32)
    h = jnp.maximum(h + b0_ref[...], 0.0)

    @pl.when(first)
    def _load_w1():
        for k in range(_W0_CHUNKS, _N_CHUNKS):
            _step(k)

    h = jnp.dot(h.astype(bf), w1_v[...], preferred_element_type=jnp.float32)
    h = jnp.maximum(h + b1_ref[...], 0.0)

    @pl.when(first)
    def _load_w2():
        pltpu.make_async_copy(w2_hbm, stage2, sem2).wait()
        def w2body(j, _):
            sl = pl.ds(j * _CVT_SUB, _CVT_SUB)
            w2_v[sl, :] = stage2[sl, :].astype(jnp.bfloat16)
            return 0
        jax.lax.fori_loop(0, N_HID // _CVT_SUB, w2body, 0, unroll=True)

    o_ref[...] = (
        jnp.dot(h.astype(bf), w2_v[...], preferred_element_type=jnp.float32)
        + b2_ref[...]
    )


def kernel(x, W0, b0, W1, b1, W2, b2):
    b0r = b0.reshape(1, N_HID)
    b1r = b1.reshape(1, N_HID)
    b2r = b2.reshape(1, N_OUT)
    grid = (BATCH // BM,)
    return pl.pallas_call(
        _mlp_body,
        grid=grid,
        in_specs=[
            pl.BlockSpec((BM, N_IN), lambda i: (i, 0)),
            pl.BlockSpec(memory_space=pl.ANY),
            pl.BlockSpec((1, N_HID), lambda i: (0, 0)),
            pl.BlockSpec(memory_space=pl.ANY),
            pl.BlockSpec((1, N_HID), lambda i: (0, 0)),
            pl.BlockSpec(memory_space=pl.ANY),
            pl.BlockSpec((1, N_OUT), lambda i: (0, 0)),
        ],
        out_specs=pl.BlockSpec((BM, N_OUT), lambda i: (i, 0)),
        out_shape=jax.ShapeDtypeStruct((BATCH, N_OUT), jnp.float32),
        scratch_shapes=[
            pltpu.VMEM((N_IN, N_HID), jnp.bfloat16),
            pltpu.VMEM((N_HID, N_HID), jnp.bfloat16),
            pltpu.VMEM((N_HID, N_OUT), jnp.bfloat16),
            pltpu.VMEM((_NSLOT, CH, N_HID), jnp.float32),
            pltpu.VMEM((N_HID, N_OUT), jnp.float32),
            pltpu.SemaphoreType.DMA((_NSLOT,)),
            pltpu.SemaphoreType.DMA,
        ],
        compiler_params=pltpu.CompilerParams(
            dimension_semantics=("arbitrary",),
        ),
    )(x, W0, b0r, W1, b1r, W2, b2r)


# speedup vs baseline: 1.0210x; 1.0008x over previous
import jax
import jax.numpy as jnp
from jax.experimental import pallas as pl
from jax.experimental.pallas import tpu as pltpu

N_IN = 3072
N_HID = 2048
N_OUT = 100
BATCH = 4096
BM = 256
NB = BATCH // BM
CH = 512
_CVT_SUB = 256
_NSLOT = 4

_W0_CHUNKS = N_IN // CH
_W1_CHUNKS = N_HID // CH


def _convert_chunk(stage_ref, slot, dst_ref, dst_row):
    def body(i, _):
        sl = pl.ds(i * _CVT_SUB, _CVT_SUB)
        dst_ref[pl.ds(dst_row + i * _CVT_SUB, _CVT_SUB), :] = (
            stage_ref[slot, sl, :].astype(jnp.bfloat16)
        )
        return 0
    jax.lax.fori_loop(0, CH // _CVT_SUB, body, 0, unroll=True)


def _mlp_body(x_ref, w0_hbm, b0_ref, w1_hbm, b1_ref, w2_hbm, b2_ref,
              o_ref, w0_v, w1_v, w2_v, h0_v, stage, stage2, sems, sem2):
    i = pl.program_id(0)

    def _w0_dma(k):
        return pltpu.make_async_copy(
            w0_hbm.at[pl.ds(k * CH, CH), :], stage.at[k % _NSLOT],
            sems.at[k % _NSLOT],
        )

    def _w1_dma(k):
        return pltpu.make_async_copy(
            w1_hbm.at[pl.ds(k * CH, CH), :], stage.at[k % _NSLOT],
            sems.at[k % _NSLOT],
        )

    @pl.when(i == 0)
    def _load_w0():
        for k in range(_NSLOT):
            _w0_dma(k).start()
        for k in range(_W0_CHUNKS):
            _w0_dma(k).wait()
            _convert_chunk(stage, k % _NSLOT, w0_v, k * CH)
            if k + _NSLOT < _W0_CHUNKS:
                _w0_dma(k + _NSLOT).start()
        for k in range(_W1_CHUNKS):
            _w1_dma(k).start()
        pltpu.make_async_copy(w2_hbm, stage2, sem2).start()

    @pl.when(i == 1)
    def _load_w1_w2():
        for k in range(_W1_CHUNKS):
            _w1_dma(k).wait()
            _convert_chunk(stage, k % _NSLOT, w1_v, k * CH)
        pltpu.make_async_copy(w2_hbm, stage2, sem2).wait()
        def w2body(j, _):
            sl = pl.ds(j * _CVT_SUB, _CVT_SUB)
            w2_v[sl, :] = stage2[sl, :].astype(jnp.bfloat16)
            return 0
        jax.lax.fori_loop(0, N_HID // _CVT_SUB, w2body, 0, unroll=True)

    bf = jnp.bfloat16
    rd = i % 2
    wr = (i + 1) % 2

    @pl.when(i > 0)
    def _layers_1_2():
        hp = h0_v[rd, :, :]
        h1 = jnp.dot(hp, w1_v[...], preferred_element_type=jnp.float32)
        h1 = jnp.maximum(h1 + b1_ref[...], 0.0)
        o_ref[...] = (
            jnp.dot(h1.astype(bf), w2_v[...],
                    preferred_element_type=jnp.float32) + b2_ref[...]
        )

    @pl.when(i < NB)
    def _layer_0():
        h0 = jnp.dot(x_ref[...].astype(bf), w0_v[...],
                     preferred_element_type=jnp.float32)
        h0_v[wr, :, :] = jnp.maximum(h0 + b0_ref[...], 0.0).astype(bf)


def kernel(x, W0, b0, W1, b1, W2, b2):
    b0r = b0.reshape(1, N_HID)
    b1r = b1.reshape(1, N_HID)
    b2r = b2.reshape(1, N_OUT)
    grid = (NB + 1,)
    return pl.pallas_call(
        _mlp_body,
        grid=grid,
        in_specs=[
            pl.BlockSpec((BM, N_IN), lambda i: (jnp.minimum(i, NB - 1), 0)),
            pl.BlockSpec(memory_space=pl.ANY),
            pl.BlockSpec((1, N_HID), lambda i: (0, 0)),
            pl.BlockSpec(memory_space=pl.ANY),
            pl.BlockSpec((1, N_HID), lambda i: (0, 0)),
            pl.BlockSpec(memory_space=pl.ANY),
            pl.BlockSpec((1, N_OUT), lambda i: (0, 0)),
        ],
        out_specs=pl.BlockSpec(
            (BM, N_OUT), lambda i: (jnp.maximum(i - 1, 0), 0)
        ),
        out_shape=jax.ShapeDtypeStruct((BATCH, N_OUT), jnp.float32),
        scratch_shapes=[
            pltpu.VMEM((N_IN, N_HID), jnp.bfloat16),
            pltpu.VMEM((N_HID, N_HID), jnp.bfloat16),
            pltpu.VMEM((N_HID, N_OUT), jnp.bfloat16),
            pltpu.VMEM((2, BM, N_HID), jnp.bfloat16),
            pltpu.VMEM((_NSLOT, CH, N_HID), jnp.float32),
            pltpu.VMEM((N_HID, N_OUT), jnp.float32),
            pltpu.SemaphoreType.DMA((_NSLOT,)),
            pltpu.SemaphoreType.DMA,
        ],
        compiler_params=pltpu.CompilerParams(
            dimension_semantics=("arbitrary",),
        ),
    )(x, W0, b0r, W1, b1r, W2, b2r)


# speedup vs baseline: 1.0450x; 1.0235x over previous
import jax
import jax.numpy as jnp
from jax.experimental import pallas as pl
from jax.experimental.pallas import tpu as pltpu

N_IN = 3072
N_HID = 2048
N_OUT = 100
BATCH = 4096
BM = 512
CH = 512
_CVT_SUB = 256


def _convert_chunk(stage_ref, slot, dst_ref, dst_row):
    def body(i, _):
        sl = pl.ds(i * _CVT_SUB, _CVT_SUB)
        dst_ref[pl.ds(dst_row + i * _CVT_SUB, _CVT_SUB), :] = (
            stage_ref[slot, sl, :].astype(jnp.bfloat16)
        )
        return 0
    jax.lax.fori_loop(0, CH // _CVT_SUB, body, 0, unroll=True)


def _mlp_body(x_ref, w0_hbm, b0_ref, w1_hbm, b1_ref, w2_hbm, b2_ref,
              o_ref, w0_v, w1_v, w2_v, stage, stage2, sems, sem2):
    first = pl.program_id(0) == 0

    chunks = (
        [(w0_hbm, w0_v, r) for r in range(0, N_IN, CH)]
        + [(w1_hbm, w1_v, r) for r in range(0, N_HID, CH)]
    )
    n = len(chunks)

    def _dma(k):
        src, _, row = chunks[k]
        return pltpu.make_async_copy(
            src.at[pl.ds(row, CH), :], stage.at[k % 2], sems.at[k % 2]
        )

    @pl.when(first)
    def _load_weights():
        _dma(0).start()
        _dma(1).start()
        pltpu.make_async_copy(w2_hbm, stage2, sem2).start()
        for k in range(n):
            _dma(k).wait()
            _, dst, row = chunks[k]
            _convert_chunk(stage, k % 2, dst, row)
            if k + 2 < n:
                _dma(k + 2).start()
        pltpu.make_async_copy(w2_hbm, stage2, sem2).wait()
        def w2body(i, _):
            sl = pl.ds(i * _CVT_SUB, _CVT_SUB)
            w2_v[sl, :] = stage2[sl, :].astype(jnp.bfloat16)
            return 0
        jax.lax.fori_loop(0, N_HID // _CVT_SUB, w2body, 0, unroll=True)

    bf = jnp.bfloat16
    h = jnp.dot(x_ref[...].astype(bf), w0_v[...],
                preferred_element_type=jnp.float32)
    h = jnp.maximum(h + b0_ref[...], 0.0)
    h = jnp.dot(h.astype(bf), w1_v[...], preferred_element_type=jnp.float32)
    h = jnp.maximum(h + b1_ref[...], 0.0)
    o_ref[...] = (
        jnp.dot(h.astype(bf), w2_v[...], preferred_element_type=jnp.float32)
        + b2_ref[...]
    )


def kernel(x, W0, b0, W1, b1, W2, b2):
    b0r = b0.reshape(1, N_HID)
    b1r = b1.reshape(1, N_HID)
    b2r = b2.reshape(1, N_OUT)
    grid = (BATCH // BM,)
    return pl.pallas_call(
        _mlp_body,
        grid=grid,
        in_specs=[
            pl.BlockSpec((BM, N_IN), lambda i: (i, 0)),
            pl.BlockSpec(memory_space=pl.ANY),
            pl.BlockSpec((1, N_HID), lambda i: (0, 0)),
            pl.BlockSpec(memory_space=pl.ANY),
            pl.BlockSpec((1, N_HID), lambda i: (0, 0)),
            pl.BlockSpec(memory_space=pl.ANY),
            pl.BlockSpec((1, N_OUT), lambda i: (0, 0)),
        ],
        out_specs=pl.BlockSpec((BM, N_OUT), lambda i: (i, 0)),
        out_shape=jax.ShapeDtypeStruct((BATCH, N_OUT), jnp.float32),
        scratch_shapes=[
            pltpu.VMEM((N_IN, N_HID), jnp.bfloat16),
            pltpu.VMEM((N_HID, N_HID), jnp.bfloat16),
            pltpu.VMEM((N_HID, N_OUT), jnp.bfloat16),
            pltpu.VMEM((2, CH, N_HID), jnp.float32),
            pltpu.VMEM((N_HID, N_OUT), jnp.float32),
            pltpu.SemaphoreType.DMA((2,)),
            pltpu.SemaphoreType.DMA,
        ],
        compiler_params=pltpu.CompilerParams(
            dimension_semantics=("arbitrary",),
        ),
    )(x, W0, b0r, W1, b1r, W2, b2r)
